# baseline (speedup 1.0000x reference)
import functools

import jax
import jax.numpy as jnp
from jax.experimental import pallas as pl
from jax.experimental.pallas import tpu as pltpu
from jax.experimental.pallas import tpu_sc as plsc

_N_E = 8192
_E_DIM = 256
_BETA = 0.25
_T = 256
_B = 8
_L = 1024
_N_TOK = _B * _L
_GRID = _N_TOK // _T


def _prep_kernel(st_ref, emb_ref, pos_ref, neg_ref, w_ref, w2x_ref, wsum_ref):
    t = st_ref[0, 0]
    style = t * pos_ref[...] + (1.0 - t) * neg_ref[...]
    w = emb_ref[...] * style
    w_ref[...] = w
    w2x_ref[...] = -2.0 * w
    wsum_ref[...] = jnp.sum(w * w, axis=1, keepdims=True)


def _prep_pallas(style_token, embedding, positive_style, negative_style):
    blk = _N_E // 4
    return pl.pallas_call(
        _prep_kernel,
        grid=(4,),
        in_specs=[
            pl.BlockSpec(memory_space=pltpu.SMEM),
            pl.BlockSpec((blk, _E_DIM), lambda i: (i, 0)),
            pl.BlockSpec((blk, _E_DIM), lambda i: (i, 0)),
            pl.BlockSpec((blk, _E_DIM), lambda i: (i, 0)),
        ],
        out_specs=[
            pl.BlockSpec((blk, _E_DIM), lambda i: (i, 0)),
            pl.BlockSpec((blk, _E_DIM), lambda i: (i, 0)),
            pl.BlockSpec((blk, 1), lambda i: (i, 0)),
        ],
        out_shape=[
            jax.ShapeDtypeStruct((_N_E, _E_DIM), jnp.float32),
            jax.ShapeDtypeStruct((_N_E, _E_DIM), jnp.float32),
            jax.ShapeDtypeStruct((_N_E, 1), jnp.float32),
        ],
    )(style_token, embedding, positive_style, negative_style)


def _vq_kernel(z_ref, znext_ref, w2x_ref, wsum_ref,
               onehot_ref, idx_ref, cnt_ref, m2_buf, zsum_buf):
    i = pl.program_id(0)

    @pl.when(i == 0)
    def _():
        zb0 = jnp.transpose(z_ref[0], (1, 0))
        m2_buf[0] = jax.lax.dot_general(
            zb0, w2x_ref[...], (((1,), (1,)), ((), ())),
            preferred_element_type=jnp.float32)
        zsum_buf[0] = jnp.sum(zb0 * zb0, axis=1, keepdims=True)

    @pl.when(i < _GRID - 1)
    def _():
        zbn = jnp.transpose(znext_ref[0], (1, 0))
        m2_buf[(i + 1) % 2] = jax.lax.dot_general(
            zbn, w2x_ref[...], (((1,), (1,)), ((), ())),
            preferred_element_type=jnp.float32)
        zsum_buf[(i + 1) % 2] = jnp.sum(zbn * zbn, axis=1, keepdims=True)

    m2 = m2_buf[i % 2]
    zsum = zsum_buf[i % 2]
    d = (zsum + wsum_ref[...]) + m2
    dmin = jnp.min(d, axis=1, keepdims=True)
    iota = jax.lax.broadcasted_iota(jnp.int32, (_T, _N_E), 1)
    cand = jnp.where(d == dmin, iota, _N_E)
    idxc = jnp.min(cand, axis=1, keepdims=True)
    onehot = jnp.where(iota == idxc, 1.0, 0.0).astype(jnp.float32)
    onehot_ref[...] = onehot
    idx_ref[...] = idxc
    cnt_ref[...] = jax.lax.dot_general(
        jnp.ones((1, _T), jnp.float32), onehot, (((1,), (0,)), ((), ())),
        preferred_element_type=jnp.float32).reshape(1, 1, _N_E)


def _vq_pallas(z, w2x, wsum_row):
    per_b = _L // _T

    def _zmap(i):
        return (i // per_b, 0, i % per_b)

    def _zmap_next(i):
        j = jnp.minimum(i + 1, _GRID - 1)
        return (j // per_b, 0, j % per_b)

    return pl.pallas_call(
        _vq_kernel,
        grid=(_GRID,),
        in_specs=[
            pl.BlockSpec((1, _E_DIM, _T), _zmap),
            pl.BlockSpec((1, _E_DIM, _T), _zmap_next),
            pl.BlockSpec((_N_E, _E_DIM), lambda i: (0, 0)),
            pl.BlockSpec((1, _N_E), lambda i: (0, 0)),
        ],
        out_specs=[
            pl.BlockSpec((_T, _N_E), lambda i: (i, 0)),
            pl.BlockSpec((_T, 1), lambda i: (i, 0)),
            pl.BlockSpec((1, 1, _N_E), lambda i: (i, 0, 0)),
        ],
        out_shape=[
            jax.ShapeDtypeStruct((_N_TOK, _N_E), jnp.float32),
            jax.ShapeDtypeStruct((_N_TOK, 1), jnp.int32),
            jax.ShapeDtypeStruct((_GRID, 1, _N_E), jnp.float32),
        ],
        scratch_shapes=[
            pltpu.VMEM((2, _T, _N_E), jnp.float32),
            pltpu.VMEM((2, _T, 1), jnp.float32),
        ],
    )(z, z, w2x, wsum_row)


_NW = 32
_BPW = _N_TOK // _NW
_IDX_CHUNK = 128


@functools.partial(
    pl.kernel,
    mesh=plsc.VectorSubcoreMesh(core_axis_name="c", subcore_axis_name="s"),
    out_type=jax.ShapeDtypeStruct((_N_TOK, _E_DIM), jnp.float32),
    scratch_types=[
        pltpu.VMEM((_BPW // _IDX_CHUNK, _IDX_CHUNK), jnp.int32),
        pltpu.VMEM((_BPW, _E_DIM), jnp.float32),
        pltpu.SemaphoreType.DMA,
    ],
)
def _sc_gather(idx_hbm, w_hbm, out_hbm, idx_v, rows_v, sem):
    wid = jax.lax.axis_index("s") * 2 + jax.lax.axis_index("c")
    nchunk = _BPW // _IDX_CHUNK
    base = wid * nchunk
    pltpu.sync_copy(idx_hbm.at[pl.ds(base, nchunk)], idx_v)
    copies = []
    for j in range(nchunk):
        copies.append(pltpu.async_copy(
            w_hbm.at[idx_v.at[j]],
            rows_v.at[pl.ds(j * _IDX_CHUNK, _IDX_CHUNK)], sem))
    for c in copies:
        c.wait()
    pltpu.sync_copy(rows_v, out_hbm.at[pl.ds(wid * _BPW, _BPW)])


def _finish_kernel(z_ref, zq_ref, cnt_ref, zqt_ref, loss_ref, ppl_ref,
                   acc_ref):
    b = pl.program_id(0)
    zqt = jnp.transpose(zq_ref[...], (1, 0))
    zqt_ref[0] = zqt
    diff = zqt - z_ref[0]
    part = jnp.sum(diff * diff)

    @pl.when(b == 0)
    def _():
        acc_ref[0, 0] = part

    @pl.when(b > 0)
    def _():
        acc_ref[0, 0] += part

    @pl.when(b == _B - 1)
    def _():
        msq = acc_ref[0, 0] * (1.0 / (_N_TOK * _E_DIM))
        loss_ref[0, 0] = _BETA * msq + msq
        counts = jnp.sum(cnt_ref[:, 0, :], axis=0, keepdims=True)
        e_mean = counts * (1.0 / _N_E)
        ppl_ref[0, 0] = jnp.exp(-jnp.sum(e_mean * jnp.log(e_mean + 1e-10)))


def _finish_pallas(z, zq_flat, cnt_rows):
    return pl.pallas_call(
        _finish_kernel,
        grid=(_B,),
        in_specs=[
            pl.BlockSpec((1, _E_DIM, _L), lambda b: (b, 0, 0)),
            pl.BlockSpec((_L, _E_DIM), lambda b: (b, 0)),
            pl.BlockSpec((_GRID, 1, _N_E), lambda b: (0, 0, 0)),
        ],
        out_specs=[
            pl.BlockSpec((1, _E_DIM, _L), lambda b: (b, 0, 0)),
            pl.BlockSpec(memory_space=pltpu.SMEM),
            pl.BlockSpec(memory_space=pltpu.SMEM),
        ],
        out_shape=[
            jax.ShapeDtypeStruct((_B, _E_DIM, _L), jnp.float32),
            jax.ShapeDtypeStruct((1, 1), jnp.float32),
            jax.ShapeDtypeStruct((1, 1), jnp.float32),
        ],
        scratch_shapes=[pltpu.SMEM((1, 1), jnp.float32)],
    )(z, zq_flat, cnt_rows)


def kernel(z, style_token, embedding, positive_style, negative_style):
    w, w2x, wsum_col = _prep_pallas(style_token, embedding, positive_style,
                                    negative_style)
    wsum_row = wsum_col.reshape(1, _N_E)

    min_encodings, min_encoding_indices, cnt_rows = _vq_pallas(
        z, w2x, wsum_row)

    zq_flat = _sc_gather(
        min_encoding_indices.reshape(_N_TOK // _IDX_CHUNK, _IDX_CHUNK), w)

    z_q, loss2d, ppl2d = _finish_pallas(z, zq_flat, cnt_rows)
    loss = loss2d.reshape(())
    perplexity = ppl2d.reshape(())
    return (z_q, loss, perplexity, min_encodings, min_encoding_indices)


# speedup vs baseline: 1.0427x; 1.0427x over previous
import functools

import jax
import jax.numpy as jnp
from jax.experimental import pallas as pl
from jax.experimental.pallas import tpu as pltpu
from jax.experimental.pallas import tpu_sc as plsc

_N_E = 8192
_E_DIM = 256
_BETA = 0.25
_T = 256
_B = 8
_L = 1024
_N_TOK = _B * _L
_GRID = _N_TOK // _T


def _prep_kernel(st_ref, emb_ref, pos_ref, neg_ref, w_ref, w2x_ref, wsum_ref):
    t = st_ref[0, 0]
    style = t * pos_ref[...] + (1.0 - t) * neg_ref[...]
    w = emb_ref[...] * style
    w_ref[...] = w
    w2x_ref[...] = -2.0 * w
    wsum_ref[...] = jnp.sum(w * w, axis=1, keepdims=True)


def _prep_pallas(style_token, embedding, positive_style, negative_style):
    blk = _N_E // 4
    return pl.pallas_call(
        _prep_kernel,
        grid=(4,),
        in_specs=[
            pl.BlockSpec(memory_space=pltpu.SMEM),
            pl.BlockSpec((blk, _E_DIM), lambda i: (i, 0)),
            pl.BlockSpec((blk, _E_DIM), lambda i: (i, 0)),
            pl.BlockSpec((blk, _E_DIM), lambda i: (i, 0)),
        ],
        out_specs=[
            pl.BlockSpec((blk, _E_DIM), lambda i: (i, 0)),
            pl.BlockSpec((blk, _E_DIM), lambda i: (i, 0)),
            pl.BlockSpec((blk, 1), lambda i: (i, 0)),
        ],
        out_shape=[
            jax.ShapeDtypeStruct((_N_E, _E_DIM), jnp.float32),
            jax.ShapeDtypeStruct((_N_E, _E_DIM), jnp.float32),
            jax.ShapeDtypeStruct((_N_E, 1), jnp.float32),
        ],
    )(style_token, embedding, positive_style, negative_style)


def _vq_kernel(z_ref, w2x_ref, wsum_ref,
               onehot_ref, idx_ref, cnt_ref):
    zb = jnp.transpose(z_ref[0], (1, 0))
    m2 = jax.lax.dot_general(
        zb, w2x_ref[...], (((1,), (1,)), ((), ())),
        preferred_element_type=jnp.float32)
    zsum = jnp.sum(zb * zb, axis=1, keepdims=True)
    d = (zsum + wsum_ref[...]) + m2
    dmin = jnp.min(d, axis=1, keepdims=True)
    iota = jax.lax.broadcasted_iota(jnp.int32, (_T, _N_E), 1)
    cand = jnp.where(d == dmin, iota, _N_E)
    idxc = jnp.min(cand, axis=1, keepdims=True)
    onehot = jnp.where(iota == idxc, 1.0, 0.0).astype(jnp.float32)
    onehot_ref[...] = onehot
    idx_ref[...] = idxc
    cnt_ref[...] = jax.lax.dot_general(
        jnp.ones((1, _T), jnp.float32), onehot, (((1,), (0,)), ((), ())),
        preferred_element_type=jnp.float32).reshape(1, 1, _N_E)


def _vq_pallas(z, w2x, wsum_row):
    per_b = _L // _T

    def _zmap(i):
        return (i // per_b, 0, i % per_b)

    return pl.pallas_call(
        _vq_kernel,
        grid=(_GRID,),
        in_specs=[
            pl.BlockSpec((1, _E_DIM, _T), _zmap),
            pl.BlockSpec((_N_E, _E_DIM), lambda i: (0, 0)),
            pl.BlockSpec((1, _N_E), lambda i: (0, 0)),
        ],
        out_specs=[
            pl.BlockSpec((_T, _N_E), lambda i: (i, 0)),
            pl.BlockSpec((_T, 1), lambda i: (i, 0)),
            pl.BlockSpec((1, 1, _N_E), lambda i: (i, 0, 0)),
        ],
        out_shape=[
            jax.ShapeDtypeStruct((_N_TOK, _N_E), jnp.float32),
            jax.ShapeDtypeStruct((_N_TOK, 1), jnp.int32),
            jax.ShapeDtypeStruct((_GRID, 1, _N_E), jnp.float32),
        ],
    )(z, w2x, wsum_row)


_NW = 32
_BPW = _N_TOK // _NW
_IDX_CHUNK = 128


@functools.partial(
    pl.kernel,
    mesh=plsc.VectorSubcoreMesh(core_axis_name="c", subcore_axis_name="s"),
    out_type=jax.ShapeDtypeStruct((_N_TOK, _E_DIM), jnp.float32),
    scratch_types=[
        pltpu.VMEM((_BPW // _IDX_CHUNK, _IDX_CHUNK), jnp.int32),
        pltpu.VMEM((_BPW, _E_DIM), jnp.float32),
        pltpu.SemaphoreType.DMA,
    ],
)
def _sc_gather(idx_hbm, w_hbm, out_hbm, idx_v, rows_v, sem):
    wid = jax.lax.axis_index("s") * 2 + jax.lax.axis_index("c")
    nchunk = _BPW // _IDX_CHUNK
    base = wid * nchunk
    pltpu.sync_copy(idx_hbm.at[pl.ds(base, nchunk)], idx_v)
    copies = []
    for j in range(nchunk):
        copies.append(pltpu.async_copy(
            w_hbm.at[idx_v.at[j]],
            rows_v.at[pl.ds(j * _IDX_CHUNK, _IDX_CHUNK)], sem))
    for c in copies:
        c.wait()
    pltpu.sync_copy(rows_v, out_hbm.at[pl.ds(wid * _BPW, _BPW)])


def _finish_kernel(z_ref, zq_ref, cnt_ref, zqt_ref, loss_ref, ppl_ref,
                   acc_ref):
    b = pl.program_id(0)
    zqt = jnp.transpose(zq_ref[...], (1, 0))
    zqt_ref[0] = zqt
    diff = zqt - z_ref[0]
    part = jnp.sum(diff * diff)

    @pl.when(b == 0)
    def _():
        acc_ref[0, 0] = part

    @pl.when(b > 0)
    def _():
        acc_ref[0, 0] += part

    @pl.when(b == _B - 1)
    def _():
        msq = acc_ref[0, 0] * (1.0 / (_N_TOK * _E_DIM))
        loss_ref[0, 0] = _BETA * msq + msq
        counts = jnp.sum(cnt_ref[:, 0, :], axis=0, keepdims=True)
        e_mean = counts * (1.0 / _N_E)
        ppl_ref[0, 0] = jnp.exp(-jnp.sum(e_mean * jnp.log(e_mean + 1e-10)))


def _finish_pallas(z, zq_flat, cnt_rows):
    return pl.pallas_call(
        _finish_kernel,
        grid=(_B,),
        in_specs=[
            pl.BlockSpec((1, _E_DIM, _L), lambda b: (b, 0, 0)),
            pl.BlockSpec((_L, _E_DIM), lambda b: (b, 0)),
            pl.BlockSpec((_GRID, 1, _N_E), lambda b: (0, 0, 0)),
        ],
        out_specs=[
            pl.BlockSpec((1, _E_DIM, _L), lambda b: (b, 0, 0)),
            pl.BlockSpec(memory_space=pltpu.SMEM),
            pl.BlockSpec(memory_space=pltpu.SMEM),
        ],
        out_shape=[
            jax.ShapeDtypeStruct((_B, _E_DIM, _L), jnp.float32),
            jax.ShapeDtypeStruct((1, 1), jnp.float32),
            jax.ShapeDtypeStruct((1, 1), jnp.float32),
        ],
        scratch_shapes=[pltpu.SMEM((1, 1), jnp.float32)],
    )(z, zq_flat, cnt_rows)


def kernel(z, style_token, embedding, positive_style, negative_style):
    w, w2x, wsum_col = _prep_pallas(style_token, embedding, positive_style,
                                    negative_style)
    wsum_row = wsum_col.reshape(1, _N_E)

    min_encodings, min_encoding_indices, cnt_rows = _vq_pallas(
        z, w2x, wsum_row)

    zq_flat = _sc_gather(
        min_encoding_indices.reshape(_N_TOK // _IDX_CHUNK, _IDX_CHUNK), w)

    z_q, loss2d, ppl2d = _finish_pallas(z, zq_flat, cnt_rows)
    loss = loss2d.reshape(())
    perplexity = ppl2d.reshape(())
    return (z_q, loss, perplexity, min_encodings, min_encoding_indices)
